# baseline (speedup 1.0000x reference)
import functools

import jax
import jax.numpy as jnp
from jax import lax
from jax.experimental import pallas as pl
from jax.experimental.pallas import tpu as pltpu
from jax.experimental.pallas import tpu_sc as plsc

BN = 512
BK = 2048


def _mlp_body(relu, out_bf16, a_ref, w_ref, b_ref, o_ref):
    a = a_ref[...].astype(jnp.bfloat16)
    w = w_ref[...].astype(jnp.bfloat16)
    o = jnp.dot(a, w, preferred_element_type=jnp.float32) + b_ref[...][None, :]
    if relu:
        o = jnp.maximum(o, 0.0)
    o_ref[...] = o.astype(jnp.bfloat16) if out_bf16 else o


def _mlp_layer(a, w, b, relu, out_bf16):
    n, kdim = a.shape
    m = w.shape[1]
    return pl.pallas_call(
        functools.partial(_mlp_body, relu, out_bf16),
        grid=(n // BN,),
        in_specs=[
            pl.BlockSpec((BN, kdim), lambda i: (i, 0)),
            pl.BlockSpec((kdim, m), lambda i: (0, 0)),
            pl.BlockSpec((m,), lambda i: (0,)),
        ],
        out_specs=pl.BlockSpec((BN, m), lambda i: (i, 0)),
        out_shape=jax.ShapeDtypeStruct((n, m), jnp.bfloat16 if out_bf16
                                       else jnp.float32),
    )(a, w, b)


def _vq_body(z_ref, cb_ref, idx_ref):
    z = z_ref[...]
    zb = z.astype(jnp.bfloat16)
    z2 = jnp.sum(z * z, axis=1, keepdims=True)
    k_tiles = cb_ref.shape[0] // BK
    iota = lax.broadcasted_iota(jnp.int32, (BN, BK), 1)

    def kbody(kt, carry):
        mv, mi = carry
        cb_f = cb_ref[pl.ds(kt * BK, BK), :]
        c2_t = jnp.sum(cb_f * cb_f, axis=1)[None, :]
        cb_t = cb_f.astype(jnp.bfloat16)
        cross = lax.dot_general(zb, cb_t, (((1,), (1,)), ((), ())),
                                preferred_element_type=jnp.float32)
        d = (z2 + c2_t) - 2.0 * cross
        tmin = jnp.min(d, axis=1, keepdims=True)
        gidx = iota + kt * BK
        tidx = jnp.min(jnp.where(d == tmin, gidx, jnp.int32(2 ** 30)),
                       axis=1, keepdims=True)
        upd = tmin < mv
        return jnp.where(upd, tmin, mv), jnp.where(upd, tidx, mi)

    mv0 = jnp.full((BN, 1), jnp.inf, jnp.float32)
    mi0 = jnp.zeros((BN, 1), jnp.int32)
    _, mi = lax.fori_loop(0, k_tiles, kbody, (mv0, mi0))
    idx_ref[...] = mi[:, 0]


def _vq_argmin(z, codebook):
    n = z.shape[0]
    k, d_code = codebook.shape
    return pl.pallas_call(
        _vq_body,
        grid=(n // BN,),
        in_specs=[
            pl.BlockSpec((BN, d_code), lambda i: (i, 0)),
            pl.BlockSpec((k, d_code), lambda i: (0, 0)),
        ],
        out_specs=pl.BlockSpec((BN,), lambda i: (i,)),
        out_shape=jax.ShapeDtypeStruct((n,), jnp.int32),
    )(z, codebook)


def _sc_gather(table, idx):
    v, d = table.shape
    b = idx.shape[0]
    info = plsc.get_sparse_core_info()
    nw = info.num_cores * info.num_subcores
    b_per_w = b // nw
    ch = 256
    nch = b_per_w // ch
    mesh = plsc.VectorSubcoreMesh(core_axis_name="c", subcore_axis_name="s")

    @functools.partial(
        pl.kernel, mesh=mesh,
        out_type=jax.ShapeDtypeStruct((b, d), jnp.float32),
        scratch_types=[
            pltpu.VMEM((b_per_w,), jnp.int32),
            pltpu.VMEM((ch, d), jnp.float32),
            pltpu.SemaphoreType.DMA,
        ],
    )
    def k(table_hbm, idx_hbm, out_hbm, idx_v, rows_v, sem):
        wid = lax.axis_index("s") * info.num_cores + lax.axis_index("c")
        base = wid * b_per_w
        pltpu.sync_copy(idx_hbm.at[pl.ds(base, b_per_w)], idx_v)
        for c in range(nch):
            pltpu.async_copy(
                table_hbm.at[idx_v.at[pl.ds(c * ch, ch)]], rows_v, sem).wait()
            pltpu.sync_copy(rows_v, out_hbm.at[pl.ds(base + c * ch, ch)])

    return k(table, idx)


def _dec_body(z_ref, q_ref, w3_ref, b3_ref, w4_ref, b4_ref,
              recon_ref, loss_ref):
    i = pl.program_id(0)
    q = q_ref[...]
    h = jnp.dot(q.astype(jnp.bfloat16), w3_ref[...].astype(jnp.bfloat16),
                preferred_element_type=jnp.float32)
    h = jnp.maximum(h + b3_ref[...][None, :], 0.0)
    r = jnp.dot(h.astype(jnp.bfloat16), w4_ref[...].astype(jnp.bfloat16),
                preferred_element_type=jnp.float32)
    recon_ref[...] = r + b4_ref[...][None, :]
    diff = z_ref[...] - q
    part = jnp.sum(diff * diff)

    @pl.when(i == 0)
    def _():
        loss_ref[0, 0] = 0.0

    loss_ref[0, 0] += part


def _dec(z, q, W3, b3, W4, b4):
    n, d_code = z.shape
    h = W3.shape[1]
    d_out = W4.shape[1]
    return pl.pallas_call(
        _dec_body,
        grid=(n // BN,),
        in_specs=[
            pl.BlockSpec((BN, d_code), lambda i: (i, 0)),
            pl.BlockSpec((BN, d_code), lambda i: (i, 0)),
            pl.BlockSpec((d_code, h), lambda i: (0, 0)),
            pl.BlockSpec((h,), lambda i: (0,)),
            pl.BlockSpec((h, d_out), lambda i: (0, 0)),
            pl.BlockSpec((d_out,), lambda i: (0,)),
        ],
        out_specs=[
            pl.BlockSpec((BN, d_out), lambda i: (i, 0)),
            pl.BlockSpec(memory_space=pltpu.SMEM),
        ],
        out_shape=[
            jax.ShapeDtypeStruct((n, d_out), jnp.float32),
            jax.ShapeDtypeStruct((1, 1), jnp.float32),
        ],
    )(z, q, W3, b3, W4, b4)


def kernel(x, W1, b1, W2, b2, codebook, W3, b3, W4, b4):
    bsz, t, d_in = x.shape
    n = bsz * t
    d_code = codebook.shape[1]
    xf = x.reshape(n, d_in)
    hb = _mlp_layer(xf, W1, b1, relu=True, out_bf16=True)
    z = _mlp_layer(hb, W2, b2, relu=False, out_bf16=False)
    idx = _vq_argmin(z, codebook)
    q = _sc_gather(codebook, idx)
    recon_f, loss = _dec(z, q, W3, b3, W4, b4)
    recon = recon_f.reshape(bsz, t, d_in)
    commit_loss = loss[0, 0] / jnp.float32(n * d_code)
    return recon, idx.reshape(bsz, t), commit_loss
